# speedup vs baseline: 1.8246x; 1.8246x over previous
import functools

import jax
import jax.numpy as jnp
from jax import lax
from jax.experimental import pallas as pl
from jax.experimental.pallas import tpu as pltpu
from jax.experimental.pallas import tpu_sc as plsc

V = 8192
BF = 8192
NC = 2
NS = 16
NW = NC * NS
BPW = BF // NW
C = 8
NCHUNK = BPW // C

_mesh = plsc.VectorSubcoreMesh(core_axis_name="c", subcore_axis_name="s")


@functools.partial(
    pl.kernel,
    mesh=_mesh,
    out_type=jax.ShapeDtypeStruct((BF, V), jnp.float32),
    scratch_types=[
        pltpu.VMEM((BPW,), jnp.int32),
        pltpu.VMEM((C, V), jnp.float32),
        pltpu.SemaphoreType.DMA,
    ],
)
def _gather_kernel(idx_hbm, table_hbm, out_hbm, idx_v, rows_v, sem):
    wid = lax.axis_index("s") * NC + lax.axis_index("c")
    base = wid * BPW
    pltpu.sync_copy(idx_hbm.at[pl.ds(base, BPW)], idx_v)

    def body(g, carry):
        pltpu.async_copy(
            table_hbm.at[idx_v.at[pl.ds(g * C, C)]], rows_v, sem
        ).wait()
        pltpu.sync_copy(rows_v, out_hbm.at[pl.ds(base + g * C, C)])
        return carry

    lax.fori_loop(0, NCHUNK, body, 0)


def kernel(idx, table):
    out = _gather_kernel(idx.reshape(-1), table)
    return out.reshape(idx.shape + (V,))


# speedup vs baseline: 1.9417x; 1.0642x over previous
import functools

import jax
import jax.numpy as jnp
from jax import lax
from jax.experimental import pallas as pl
from jax.experimental.pallas import tpu as pltpu
from jax.experimental.pallas import tpu_sc as plsc

V = 8192
BF = 8192
NC = 2
NS = 16
NW = NC * NS
BPW = BF // NW
C = 2
NB = 4
NCHUNK = BPW // C
NOUT = NCHUNK // NB

_mesh = plsc.VectorSubcoreMesh(core_axis_name="c", subcore_axis_name="s")


@functools.partial(
    pl.kernel,
    mesh=_mesh,
    out_type=jax.ShapeDtypeStruct((BF, V), jnp.float32),
    scratch_types=[
        pltpu.VMEM((NCHUNK, C), jnp.int32),
        pltpu.VMEM((C, V), jnp.float32),
        pltpu.VMEM((C, V), jnp.float32),
        pltpu.VMEM((C, V), jnp.float32),
        pltpu.VMEM((C, V), jnp.float32),
        pltpu.SemaphoreType.DMA,
        pltpu.SemaphoreType.DMA,
        pltpu.SemaphoreType.DMA,
        pltpu.SemaphoreType.DMA,
        pltpu.SemaphoreType.DMA,
        pltpu.SemaphoreType.DMA,
        pltpu.SemaphoreType.DMA,
        pltpu.SemaphoreType.DMA,
    ],
)
def _gather_kernel(idx_hbm, table_hbm, out_hbm, idx_v,
                   r0, r1, r2, r3, gs0, gs1, gs2, gs3, ws0, ws1, ws2, ws3):
    wid = lax.axis_index("s") * NC + lax.axis_index("c")
    base = wid * BPW
    pltpu.sync_copy(idx_hbm.at[wid], idx_v)

    bufs = [r0, r1, r2, r3]
    gsems = [gs0, gs1, gs2, gs3]
    wsems = [ws0, ws1, ws2, ws3]

    def gcopy(g, b):
        return pltpu.make_async_copy(
            table_hbm.at[idx_v.at[g]], bufs[b], gsems[b])

    def wcopy(g, b):
        return pltpu.make_async_copy(
            bufs[b], out_hbm.at[pl.ds(base + g * C, C)], wsems[b])

    def gstart(g, b):
        gcopy(g, b).start()

    def gwait(g, b):
        gcopy(g, b).wait()

    def wstart(g, b):
        wcopy(g, b).start()

    def wwait(g, b):
        wcopy(g, b).wait()

    gstart(0, 0)
    gstart(1, 1)
    gwait(0, 0); wstart(0, 0); gstart(2, 2)
    gwait(1, 1); wstart(1, 1); gstart(3, 3)
    gwait(2, 2); wstart(2, 2); wwait(0, 0); gstart(4, 0)
    gwait(3, 3); wstart(3, 3); wwait(1, 1); gstart(5, 1)

    def body(o, carry):
        g0 = o * NB
        for b in range(NB):
            g = g0 + b
            b2 = (b + 2) % NB
            gwait(g, b)
            wstart(g, b)
            wwait(g - 2, b2)
            gstart(g + 2, b2)
        return carry

    lax.fori_loop(1, NOUT - 1, body, 0)

    g0 = (NOUT - 1) * NB
    gwait(g0 + 0, 0); wstart(g0 + 0, 0); wwait(g0 - 2, 2); gstart(g0 + 2, 2)
    gwait(g0 + 1, 1); wstart(g0 + 1, 1); wwait(g0 - 1, 3); gstart(g0 + 3, 3)
    gwait(g0 + 2, 2); wstart(g0 + 2, 2)
    gwait(g0 + 3, 3); wstart(g0 + 3, 3)
    wwait(g0 + 0, 0); wwait(g0 + 1, 1); wwait(g0 + 2, 2); wwait(g0 + 3, 3)


def kernel(idx, table):
    out = _gather_kernel(idx.reshape(NW, NCHUNK, C), table)
    return out.reshape(idx.shape + (V,))
